# baseline (speedup 1.0000x reference)
import functools

import jax
import jax.numpy as jnp
from jax import lax
from jax.experimental import pallas as pl
from jax.experimental.pallas import tpu as pltpu
from jax.experimental.pallas import tpu_sc as plsc

N = 50000
M = 12500
K = 16
C = 128
OUT = 256

NC = 2
NS = 16
NW = NC * NS
PW = 400
M_PAD = NW * PW
G = 8
CHUNKS = PW // G
LANES = 16


def _ln_proj_body(x_ref, g_ref, b_ref, wt_ref, y_ref):
    xb = x_ref[...]
    mu = jnp.mean(xb, axis=1, keepdims=True)
    xc = xb - mu
    var = jnp.mean(xc * xc, axis=1, keepdims=True)
    normed = xc * lax.rsqrt(var + 1e-5) * g_ref[...] + b_ref[...]
    y = jnp.dot(normed, wt_ref[...], preferred_element_type=jnp.float32)
    lo = lax.bitcast_convert_type(y[:, : OUT // 2].astype(jnp.bfloat16), jnp.uint16)
    hi = lax.bitcast_convert_type(y[:, OUT // 2 :].astype(jnp.bfloat16), jnp.uint16)
    y_ref[...] = lo.astype(jnp.uint32) | (hi.astype(jnp.uint32) << 16)


def _ln_proj(x, gamma, beta, wt):
    BN = 1000
    grid = N // BN
    return pl.pallas_call(
        _ln_proj_body,
        grid=(grid,),
        in_specs=[
            pl.BlockSpec((BN, C), lambda i: (i, 0)),
            pl.BlockSpec((1, C), lambda i: (0, 0)),
            pl.BlockSpec((1, C), lambda i: (0, 0)),
            pl.BlockSpec((C, OUT), lambda i: (0, 0)),
        ],
        out_specs=pl.BlockSpec((BN, OUT // 2), lambda i: (i, 0)),
        out_shape=jax.ShapeDtypeStruct((N, OUT // 2), jnp.uint32),
    )(x, gamma.reshape(1, C), beta.reshape(1, C), wt)


def _compute_chunk(rows_v, outb_v):
    hi_mask = jnp.uint32(0xFFFF0000)
    f32 = jnp.float32
    u32 = jnp.uint32
    bc = lax.bitcast_convert_type
    for j in range(G):
        for cv in range(OUT // (2 * LANES)):
            sl = pl.ds(cv * LANES, LANES)
            w = rows_v[j * K, sl]
            acc_lo = bc(w << 16, f32)
            acc_w = bc(w, f32)
            for r in range(1, K):
                w = rows_v[j * K + r, sl]
                acc_lo = jnp.maximum(acc_lo, bc(w << 16, f32))
                acc_w = jnp.maximum(acc_w, bc(w, f32))
            outb_v[j, sl] = (bc(acc_lo, u32) >> 16) | (bc(acc_w, u32) & hi_mask)


def _gather_max_body(y_hbm, idx_hbm, out_hbm, idx_v, rows_v, outb_v, sem):
    wid = lax.axis_index("s") * NC + lax.axis_index("c")
    base = wid * PW

    pltpu.sync_copy(idx_hbm.at[pl.ds(base * K, PW * K)], idx_v)

    def chunk_body(ci, carry):
        idx_slice = idx_v.at[pl.ds(ci * (G * K), G * K)]
        pltpu.async_copy(y_hbm.at[idx_slice], rows_v, sem).wait()
        _compute_chunk(rows_v, outb_v)
        pltpu.sync_copy(outb_v, out_hbm.at[pl.ds(base + ci * G, G)])
        return carry

    lax.fori_loop(0, CHUNKS, chunk_body, 0, unroll=False)


def _gather_max(y, idx_flat):
    mesh = plsc.VectorSubcoreMesh(core_axis_name="c", subcore_axis_name="s")
    fn = pl.kernel(
        _gather_max_body,
        out_type=jax.ShapeDtypeStruct((M_PAD, OUT // 2), jnp.uint32),
        mesh=mesh,
        scratch_types=[
            pltpu.VMEM((PW * K,), jnp.int32),
            pltpu.VMEM((G * K, OUT // 2), jnp.uint32),
            pltpu.VMEM((G, OUT // 2), jnp.uint32),
            pltpu.SemaphoreType.DMA,
        ],
    )
    return fn(y, idx_flat)


def kernel(p, x, o, n_p, knn_idx, n_o, gamma, beta, W):
    y = _ln_proj(x, gamma, beta, W.T)

    idx32 = knn_idx.astype(jnp.int32)
    idx_pad = jnp.zeros((M_PAD, K), jnp.int32).at[:M].set(idx32)
    u = _gather_max(y, idx_pad.reshape(M_PAD * K))[:M]
    lo = lax.bitcast_convert_type((u & 0xFFFF).astype(jnp.uint16), jnp.bfloat16)
    hi = lax.bitcast_convert_type((u >> 16).astype(jnp.uint16), jnp.bfloat16)
    out = jnp.concatenate(
        [lo.astype(jnp.float32), hi.astype(jnp.float32)], axis=1)
    return (out, n_p, n_o)


# speedup vs baseline: 1.8205x; 1.8205x over previous
import functools

import jax
import jax.numpy as jnp
from jax import lax
from jax.experimental import pallas as pl
from jax.experimental.pallas import tpu as pltpu
from jax.experimental.pallas import tpu_sc as plsc

N = 50000
M = 12500
K = 16
C = 128
OUT = 256

NC = 2
NS = 16
NW = NC * NS
PW = 392
M_PAD = NW * PW
G = 8
CHUNKS = PW // G
LANES = 16


def _ln_proj_body(x_ref, g_ref, b_ref, wt_ref, y_ref):
    xb = x_ref[...]
    mu = jnp.mean(xb, axis=1, keepdims=True)
    xc = xb - mu
    var = jnp.mean(xc * xc, axis=1, keepdims=True)
    normed = xc * lax.rsqrt(var + 1e-5) * g_ref[...] + b_ref[...]
    y = jnp.dot(normed, wt_ref[...], preferred_element_type=jnp.float32)
    lo = lax.bitcast_convert_type(y[:, : OUT // 2].astype(jnp.bfloat16), jnp.uint16)
    hi = lax.bitcast_convert_type(y[:, OUT // 2 :].astype(jnp.bfloat16), jnp.uint16)
    y_ref[...] = lo.astype(jnp.uint32) | (hi.astype(jnp.uint32) << 16)


def _ln_proj(x, gamma, beta, wt):
    BN = 1000
    grid = N // BN
    return pl.pallas_call(
        _ln_proj_body,
        grid=(grid,),
        in_specs=[
            pl.BlockSpec((BN, C), lambda i: (i, 0)),
            pl.BlockSpec((1, C), lambda i: (0, 0)),
            pl.BlockSpec((1, C), lambda i: (0, 0)),
            pl.BlockSpec((C, OUT), lambda i: (0, 0)),
        ],
        out_specs=pl.BlockSpec((BN, OUT // 2), lambda i: (i, 0)),
        out_shape=jax.ShapeDtypeStruct((N, OUT // 2), jnp.uint32),
    )(x, gamma.reshape(1, C), beta.reshape(1, C), wt)


def _compute_chunk(rows_v, outb_v):
    hi_mask = jnp.uint32(0xFFFF0000)
    f32 = jnp.float32
    u32 = jnp.uint32
    bc = lax.bitcast_convert_type
    for j in range(G):
        for cv in range(OUT // (2 * LANES)):
            sl = pl.ds(cv * LANES, LANES)
            w = rows_v[j * K, sl]
            acc_lo = bc(w << 16, f32)
            acc_hi = bc(w & hi_mask, f32)
            for r in range(1, K):
                w = rows_v[j * K + r, sl]
                acc_lo = jnp.maximum(acc_lo, bc(w << 16, f32))
                acc_hi = jnp.maximum(acc_hi, bc(w & hi_mask, f32))
            outb_v[j, sl] = (bc(acc_lo, u32) >> 16) | bc(acc_hi, u32)


def _gather_max_body(y_hbm, idx_hbm, out_hbm, idx_v, rows_v, outb_v, sem):
    wid = lax.axis_index("s") * NC + lax.axis_index("c")
    base = wid * PW

    pltpu.sync_copy(idx_hbm.at[pl.ds(base * K, PW * K)], idx_v)

    def chunk_body(ci, carry):
        idx_slice = idx_v.at[pl.ds(ci * (G * K), G * K)]
        pltpu.async_copy(y_hbm.at[idx_slice], rows_v, sem).wait()
        _compute_chunk(rows_v, outb_v)
        pltpu.sync_copy(outb_v, out_hbm.at[pl.ds(base + ci * G, G)])
        return carry

    lax.fori_loop(0, CHUNKS, chunk_body, 0, unroll=False)


def _gather_max(y, idx_flat):
    mesh = plsc.VectorSubcoreMesh(core_axis_name="c", subcore_axis_name="s")
    fn = pl.kernel(
        _gather_max_body,
        out_type=jax.ShapeDtypeStruct((M_PAD, OUT // 2), jnp.uint32),
        mesh=mesh,
        scratch_types=[
            pltpu.VMEM((PW * K,), jnp.int32),
            pltpu.VMEM((G * K, OUT // 2), jnp.uint32),
            pltpu.VMEM((G, OUT // 2), jnp.uint32),
            pltpu.SemaphoreType.DMA,
        ],
    )
    return fn(y, idx_flat)


def kernel(p, x, o, n_p, knn_idx, n_o, gamma, beta, W):
    y = _ln_proj(x, gamma, beta, W.T)

    idx32 = knn_idx.astype(jnp.int32)
    idx_pad = jnp.zeros((M_PAD, K), jnp.int32).at[:M].set(idx32)
    u = _gather_max(y, idx_pad.reshape(M_PAD * K))[:M]
    lo = lax.bitcast_convert_type((u & 0xFFFF).astype(jnp.uint16), jnp.bfloat16)
    hi = lax.bitcast_convert_type((u >> 16).astype(jnp.uint16), jnp.bfloat16)
    out = jnp.concatenate(
        [lo.astype(jnp.float32), hi.astype(jnp.float32)], axis=1)
    return (out, n_p, n_o)


# speedup vs baseline: 1.8855x; 1.0357x over previous
import functools

import jax
import jax.numpy as jnp
from jax import lax
from jax.experimental import pallas as pl
from jax.experimental.pallas import tpu as pltpu
from jax.experimental.pallas import tpu_sc as plsc

N = 50000
M = 12500
K = 16
C = 128
OUT = 256

NC = 2
NS = 16
NW = NC * NS
PW = 392
M_PAD = NW * PW
G = 8
CHUNKS = PW // G
LANES = 16


def _ln_proj_body(x_ref, g_ref, b_ref, wt_ref, y_ref):
    xb = x_ref[...]
    mu = jnp.mean(xb, axis=1, keepdims=True)
    xc = xb - mu
    var = jnp.mean(xc * xc, axis=1, keepdims=True)
    normed = xc * lax.rsqrt(var + 1e-5) * g_ref[...] + b_ref[...]
    y = jnp.dot(normed, wt_ref[...], preferred_element_type=jnp.float32)
    lo = lax.bitcast_convert_type(y[:, : OUT // 2].astype(jnp.bfloat16), jnp.uint16)
    hi = lax.bitcast_convert_type(y[:, OUT // 2 :].astype(jnp.bfloat16), jnp.uint16)
    y_ref[...] = lo.astype(jnp.uint32) | (hi.astype(jnp.uint32) << 16)


def _ln_proj(x, gamma, beta, wt):
    BN = 1000
    grid = N // BN
    return pl.pallas_call(
        _ln_proj_body,
        grid=(grid,),
        in_specs=[
            pl.BlockSpec((BN, C), lambda i: (i, 0)),
            pl.BlockSpec((1, C), lambda i: (0, 0)),
            pl.BlockSpec((1, C), lambda i: (0, 0)),
            pl.BlockSpec((C, OUT), lambda i: (0, 0)),
        ],
        out_specs=pl.BlockSpec((BN, OUT // 2), lambda i: (i, 0)),
        out_shape=jax.ShapeDtypeStruct((N, OUT // 2), jnp.uint32),
    )(x, gamma.reshape(1, C), beta.reshape(1, C), wt)


def _compute_chunk(rows_v, outb_v):
    hi_mask = jnp.uint32(0xFFFF0000)
    f32 = jnp.float32
    u32 = jnp.uint32
    bc = lax.bitcast_convert_type
    for j in range(G):
        for cv in range(OUT // (2 * LANES)):
            sl = pl.ds(cv * LANES, LANES)
            w = rows_v[j * K, sl]
            acc_lo = bc(w << 16, f32)
            acc_w = bc(w, f32)
            for r in range(1, K):
                w = rows_v[j * K + r, sl]
                acc_lo = jnp.maximum(acc_lo, bc(w << 16, f32))
                acc_w = jnp.maximum(acc_w, bc(w, f32))
            outb_v[j, sl] = (bc(acc_lo, u32) >> 16) | (bc(acc_w, u32) & hi_mask)


def _gather_max_body(y_hbm, idx_hbm, out_hbm, idx_v, rows_v, outb_v, sem):
    wid = lax.axis_index("s") * NC + lax.axis_index("c")
    base = wid * PW

    pltpu.sync_copy(idx_hbm.at[pl.ds(base * K, PW * K)], idx_v)

    def chunk_body(ci, carry):
        idx_slice = idx_v.at[pl.ds(ci * (G * K), G * K)]
        pltpu.async_copy(y_hbm.at[idx_slice], rows_v, sem).wait()
        _compute_chunk(rows_v, outb_v)
        pltpu.sync_copy(outb_v, out_hbm.at[pl.ds(base + ci * G, G)])
        return carry

    lax.fori_loop(0, CHUNKS, chunk_body, 0, unroll=False)


def _gather_max(y, idx_flat):
    mesh = plsc.VectorSubcoreMesh(core_axis_name="c", subcore_axis_name="s")
    fn = pl.kernel(
        _gather_max_body,
        out_type=jax.ShapeDtypeStruct((M_PAD, OUT // 2), jnp.uint32),
        mesh=mesh,
        scratch_types=[
            pltpu.VMEM((PW * K,), jnp.int32),
            pltpu.VMEM((G * K, OUT // 2), jnp.uint32),
            pltpu.VMEM((G, OUT // 2), jnp.uint32),
            pltpu.SemaphoreType.DMA,
        ],
    )
    return fn(y, idx_flat)


def kernel(p, x, o, n_p, knn_idx, n_o, gamma, beta, W):
    y = _ln_proj(x, gamma, beta, W.T)

    idx32 = knn_idx.astype(jnp.int32)
    idx_pad = jnp.zeros((M_PAD, K), jnp.int32).at[:M].set(idx32)
    u = _gather_max(y, idx_pad.reshape(M_PAD * K))[:M]
    lo = lax.bitcast_convert_type((u & 0xFFFF).astype(jnp.uint16), jnp.bfloat16)
    hi = lax.bitcast_convert_type((u >> 16).astype(jnp.uint16), jnp.bfloat16)
    out = jnp.concatenate(
        [lo.astype(jnp.float32), hi.astype(jnp.float32)], axis=1)
    return (out, n_p, n_o)
